# baseline (speedup 1.0000x reference)
import functools

import jax
import jax.numpy as jnp
from jax import lax
from jax.experimental import pallas as pl
from jax.experimental.pallas import tpu as pltpu
from jax.experimental.pallas import tpu_sc as plsc

B = 16384
F = 26
V = 1000000

NC, NS, L = 2, 16, 16
NW = NC * NS
RPW = B // NW
EPW = RPW * F


def _body(x_hbm, tab_hbm, out_hbm, idx_v, g_v, o_v, sem):
    wid = lax.axis_index("s") * NC + lax.axis_index("c")
    pltpu.sync_copy(x_hbm.at[pl.ds(wid * EPW, EPW)], idx_v)

    lane = lax.iota(jnp.int32, L)

    def idx_body(j, _):
        pos = lane + j * L
        f = lax.rem(pos, F)
        idx_v[pl.ds(j * L, L)] = idx_v[pl.ds(j * L, L)] + f * V
        return 0

    lax.fori_loop(0, EPW // L, idx_body, 0)

    pltpu.async_copy(tab_hbm.at[idx_v], g_v, sem).wait()

    def red_body(j, _):
        p = (lane + j * L) * F
        acc = plsc.load_gather(g_v, [p])
        for f in range(1, F):
            acc = acc + plsc.load_gather(g_v, [p + f])
        o_v[pl.ds(j * L, L)] = acc
        return 0

    lax.fori_loop(0, RPW // L, red_body, 0)

    pltpu.sync_copy(o_v, out_hbm.at[pl.ds(wid * RPW, RPW)])


@jax.jit
def kernel(X, tables):
    x_flat = X.reshape(B * F)
    tab_flat = tables.reshape(F * V)
    run = functools.partial(
        pl.kernel,
        out_type=jax.ShapeDtypeStruct((B,), jnp.float32),
        mesh=plsc.VectorSubcoreMesh(core_axis_name="c", subcore_axis_name="s"),
        scratch_types=[
            pltpu.VMEM((EPW,), jnp.int32),
            pltpu.VMEM((EPW,), jnp.float32),
            pltpu.VMEM((RPW,), jnp.float32),
            pltpu.SemaphoreType.DMA,
        ],
        compiler_params=pltpu.CompilerParams(needs_layout_passes=False),
    )(_body)
    out = run(x_flat, tab_flat)
    return out.reshape(B, 1)
